# Initial kernel scaffold; baseline (speedup 1.0000x reference)
import jax
import jax.numpy as jnp
from jax.experimental import pallas as pl


def kernel(x, edge_index, Wsrc, bsrc, Wdst, bdst, attn, gat_bias, W1, b1, W2, b2):
    raise NotImplementedError("write your pallas kernel here")


# speedup vs baseline: 4.3725x; 4.3725x over previous
import functools

import jax
import jax.numpy as jnp
from jax import lax
from jax.experimental import pallas as pl
from jax.experimental.pallas import tpu as pltpu
from jax.experimental.pallas import tpu_sc as plsc

NC = 2
NS = 16
NW = NC * NS
LANES = 16


def _proj_body(x_ref, ws_ref, bs_ref, wd_ref, bd_ref, fs_ref, fd_ref):
    x = x_ref[...]
    fs_ref[...] = jnp.dot(x, ws_ref[...],
                          preferred_element_type=jnp.float32) + bs_ref[...]
    fd_ref[...] = jnp.dot(x, wd_ref[...],
                          preferred_element_type=jnp.float32) + bd_ref[...]


def _proj(x, Ws, bs, Wd, bd):
    n, d = x.shape
    return pl.pallas_call(
        _proj_body,
        out_shape=[jax.ShapeDtypeStruct((n, d), jnp.float32),
                   jax.ShapeDtypeStruct((n, d), jnp.float32)],
    )(x, Ws, bs, Wd, bd)


def _make_pass1(N, E, D, NPAD, C):
    EW = E // NW
    NCHUNK = EW // C
    G = C // LANES
    mesh = plsc.VectorSubcoreMesh(core_axis_name="c", subcore_axis_name="s",
                                  num_cores=NC, num_subcores=NS)

    @functools.partial(
        pl.kernel,
        out_type=[jax.ShapeDtypeStruct((E,), jnp.float32),
                  jax.ShapeDtypeStruct((NW, NPAD), jnp.float32)],
        mesh=mesh,
        compiler_params=pltpu.CompilerParams(needs_layout_passes=False),
        scratch_types=[
            pltpu.VMEM((C,), jnp.int32),
            pltpu.VMEM((C,), jnp.int32),
            pltpu.VMEM((C, D), jnp.float32),
            pltpu.VMEM((C, D), jnp.float32),
            pltpu.VMEM((C,), jnp.float32),
            pltpu.VMEM((NPAD,), jnp.float32),
            pltpu.VMEM((D,), jnp.float32),
            pltpu.SemaphoreType.DMA,
        ],
    )
    def pass1(fs_hbm, fd_hbm, esrc_hbm, edst_hbm, attn_hbm, ex_hbm,
              den_all_hbm,
              src_v, dst_v, fsrows, fdrows, ex_v, den_priv, attn_v, sem):
        wid = lax.axis_index("s") * NC + lax.axis_index("c")
        woff = wid * EW
        pltpu.sync_copy(attn_hbm, attn_v)

        def zero_body(i, _):
            den_priv[pl.ds(i * LANES, LANES)] = jnp.zeros((LANES,), jnp.float32)
            return 0
        lax.fori_loop(0, NPAD // LANES, zero_body, 0)

        iota = lax.iota(jnp.int32, LANES)

        def chunk_body(c, _):
            off = woff + c * C
            pltpu.sync_copy(esrc_hbm.at[pl.ds(off, C)], src_v)
            pltpu.sync_copy(edst_hbm.at[pl.ds(off, C)], dst_v)
            cp1 = pltpu.async_copy(fs_hbm.at[src_v], fsrows, sem)
            cp2 = pltpu.async_copy(fd_hbm.at[dst_v], fdrows, sem)
            cp1.wait()
            cp2.wait()
            for g in range(G):
                rows = iota + g * LANES
                dstg = dst_v[pl.ds(g * LANES, LANES)]

                def dot_body(k, acc, rows=rows):
                    ach = attn_v[pl.ds(k * LANES, LANES)]
                    dbase = k * LANES
                    for j in range(LANES):
                        a = ach[j]
                        dcol = jnp.broadcast_to(dbase + j, (LANES,))
                        vs = plsc.load_gather(fsrows, [rows, dcol])
                        vd = plsc.load_gather(fdrows, [rows, dcol])
                        u = vs + vd
                        acc = acc + u * (0.6 * a) + jnp.abs(u) * (0.4 * a)
                    return acc

                acc = lax.fori_loop(0, D // LANES, dot_body,
                                    jnp.zeros((LANES,), jnp.float32))
                exv = jnp.exp(acc)
                ex_v[pl.ds(g * LANES, LANES)] = exv
                plsc.addupdate_scatter(den_priv, [dstg], exv)
            pltpu.sync_copy(ex_v, ex_hbm.at[pl.ds(off, C)])
            return 0

        lax.fori_loop(0, NCHUNK, chunk_body, 0)
        pltpu.sync_copy(den_priv, den_all_hbm.at[wid])

    return pass1


def _make_pass2(N, E, NPAD, C2):
    EW = E // NW
    NCHUNK = EW // C2
    mesh = plsc.VectorSubcoreMesh(core_axis_name="c", subcore_axis_name="s",
                                  num_cores=NC, num_subcores=NS)

    @functools.partial(
        pl.kernel,
        out_type=[jax.ShapeDtypeStruct((NW, NPAD), jnp.float32)],
        mesh=mesh,
        compiler_params=pltpu.CompilerParams(needs_layout_passes=False),
        scratch_types=[
            pltpu.VMEM((C2,), jnp.int32),
            pltpu.VMEM((C2,), jnp.int32),
            pltpu.VMEM((C2,), jnp.float32),
            pltpu.VMEM((NPAD,), jnp.float32),
            pltpu.VMEM((NPAD,), jnp.float32),
            pltpu.VMEM((NPAD,), jnp.float32),
            pltpu.SemaphoreType.DMA,
        ],
    )
    def pass2(ex_hbm, esrc_hbm, edst_hbm, den_all_hbm, w_all_hbm,
              src_v, dst_v, ex_v, den_v, tmp_v, w_priv, sem):
        wid = lax.axis_index("s") * NC + lax.axis_index("c")
        woff = wid * EW
        nvec = NPAD // LANES

        def zero_body(i, _):
            w_priv[pl.ds(i * LANES, LANES)] = jnp.zeros((LANES,), jnp.float32)
            den_v[pl.ds(i * LANES, LANES)] = jnp.zeros((LANES,), jnp.float32)
            return 0
        lax.fori_loop(0, nvec, zero_body, 0)

        def acc_row(r, _):
            pltpu.sync_copy(den_all_hbm.at[r], tmp_v)

            def add_body(i, _):
                sl = pl.ds(i * LANES, LANES)
                den_v[sl] = den_v[sl] + tmp_v[sl]
                return 0
            lax.fori_loop(0, nvec, add_body, 0)
            return 0
        lax.fori_loop(0, NW, acc_row, 0)

        def chunk_body(c, _):
            off = woff + c * C2
            pltpu.sync_copy(esrc_hbm.at[pl.ds(off, C2)], src_v)
            pltpu.sync_copy(edst_hbm.at[pl.ds(off, C2)], dst_v)
            pltpu.sync_copy(ex_hbm.at[pl.ds(off, C2)], ex_v)

            def grp_body(g, _):
                sl = pl.ds(g * LANES, LANES)
                dstg = dst_v[sl]
                srcg = src_v[sl]
                exg = ex_v[sl]
                dv = plsc.load_gather(den_v, [dstg])
                a = exg / (dv + 1e-16)
                plsc.addupdate_scatter(w_priv, [srcg], a)
                return 0
            lax.fori_loop(0, C2 // LANES, grp_body, 0)
            return 0

        lax.fori_loop(0, NCHUNK, chunk_body, 0)
        pltpu.sync_copy(w_priv, w_all_hbm.at[wid])

    return pass2


def _make_final(N, D, NPAD):
    def _final_body(w_all_ref, fs_ref, gb_ref, w1_ref, b1_ref, w2_ref, b2_ref,
                    out_ref):
        w = jnp.sum(w_all_ref[...], axis=0)[:N]
        pooled = jnp.sum(fs_ref[...] * w[:, None], axis=0) * (1.0 / N)
        hg = jax.nn.sigmoid(pooled + gb_ref[0])
        h1 = jnp.sum(w1_ref[...] * hg[:, None], axis=0) + b1_ref[0]
        h2 = jnp.sum(w2_ref[...] * h1[:, None], axis=0) + b2_ref[0]
        out_ref[...] = jax.nn.sigmoid(h2).reshape(1, 1)

    def _final(w_all, fs, gb, W1, b1, W2, b2):
        return pl.pallas_call(
            _final_body,
            out_shape=jax.ShapeDtypeStruct((1, 1), jnp.float32),
        )(w_all, fs, gb, W1, b1, W2, b2)

    return _final


def kernel(x, edge_index, Wsrc, bsrc, Wdst, bdst, attn, gat_bias, W1, b1,
           W2, b2):
    N, D = x.shape
    E = edge_index.shape[1]
    NPAD = 10240
    C = 80
    C2 = 2000

    fs, fd = _proj(x, Wsrc, bsrc.reshape(1, -1), Wdst, bdst.reshape(1, -1))
    esrc = edge_index[0]
    edst = edge_index[1]
    ex, den_all = _make_pass1(N, E, D, NPAD, C)(fs, fd, esrc, edst,
                                                attn.reshape(-1))
    w_all, = _make_pass2(N, E, NPAD, C2)(ex, esrc, edst, den_all)
    out = _make_final(N, D, NPAD)(w_all, fs, gat_bias.reshape(1, -1),
                                  W1, b1.reshape(1, -1), W2, b2.reshape(1, -1))
    return out.reshape(1, 1, 1)


# speedup vs baseline: 5.2662x; 1.2044x over previous
import functools

import jax
import jax.numpy as jnp
from jax import lax
from jax.experimental import pallas as pl
from jax.experimental.pallas import tpu as pltpu
from jax.experimental.pallas import tpu_sc as plsc

NC = 2
NS = 16
NW = NC * NS
LANES = 16


def _proj_body(x_ref, ws_ref, bs_ref, wd_ref, bd_ref, fs_ref, fd_ref):
    x = x_ref[...]
    fs_ref[...] = jnp.dot(x, ws_ref[...],
                          preferred_element_type=jnp.float32) + bs_ref[...]
    fd_ref[...] = jnp.dot(x, wd_ref[...],
                          preferred_element_type=jnp.float32) + bd_ref[...]


def _proj(x, Ws, bs, Wd, bd):
    n, d = x.shape
    return pl.pallas_call(
        _proj_body,
        out_shape=[jax.ShapeDtypeStruct((n, d), jnp.float32),
                   jax.ShapeDtypeStruct((n, d), jnp.float32)],
    )(x, Ws, bs, Wd, bd)


def _make_pass1(N, E, D, NPAD, C):
    EW = E // NW
    NCHUNK = EW // C
    G = C // LANES
    mesh = plsc.VectorSubcoreMesh(core_axis_name="c", subcore_axis_name="s",
                                  num_cores=NC, num_subcores=NS)

    @functools.partial(
        pl.kernel,
        out_type=[jax.ShapeDtypeStruct((E,), jnp.float32),
                  jax.ShapeDtypeStruct((NW, NPAD), jnp.float32)],
        mesh=mesh,
        compiler_params=pltpu.CompilerParams(needs_layout_passes=False),
        scratch_types=[
            pltpu.VMEM((EW,), jnp.int32),
            pltpu.VMEM((EW,), jnp.int32),
            pltpu.VMEM((C, D), jnp.float32),
            pltpu.VMEM((C, D), jnp.float32),
            pltpu.VMEM((C, D), jnp.float32),
            pltpu.VMEM((C, D), jnp.float32),
            pltpu.VMEM((EW,), jnp.float32),
            pltpu.VMEM((NPAD,), jnp.float32),
            pltpu.VMEM((D,), jnp.float32),
            pltpu.SemaphoreType.DMA,
            pltpu.SemaphoreType.DMA,
        ],
    )
    def pass1(fs_hbm, fd_hbm, esrc_hbm, edst_hbm, attn_hbm, ex_hbm,
              den_all_hbm,
              src_all, dst_all, fsr0, fdr0, fsr1, fdr1, ex_all, den_priv,
              attn_v, s0, s1):
        wid = lax.axis_index("s") * NC + lax.axis_index("c")
        woff = wid * EW
        pltpu.sync_copy(attn_hbm, attn_v)
        pltpu.sync_copy(esrc_hbm.at[pl.ds(woff, EW)], src_all)
        pltpu.sync_copy(edst_hbm.at[pl.ds(woff, EW)], dst_all)

        def zero_body(i, _):
            den_priv[pl.ds(i * LANES, LANES)] = jnp.zeros((LANES,), jnp.float32)
            return 0
        lax.fori_loop(0, NPAD // LANES, zero_body, 0)

        iota = lax.iota(jnp.int32, LANES)

        def issue(c, fsr, fdr, sem):
            sl = pl.ds(c * C, C)
            pltpu.async_copy(fs_hbm.at[src_all.at[sl]], fsr, sem)
            pltpu.async_copy(fd_hbm.at[dst_all.at[sl]], fdr, sem)

        def waitg(c, fsr, fdr, sem):
            sl = pl.ds(c * C, C)
            pltpu.make_async_copy(fs_hbm.at[src_all.at[sl]], fsr, sem).wait()
            pltpu.make_async_copy(fd_hbm.at[dst_all.at[sl]], fdr, sem).wait()

        def compute(c, fsr, fdr):
            base = c * C
            for g in range(G):
                rows = iota + g * LANES
                dstg = dst_all[pl.ds(base + g * LANES, LANES)]

                def dot_body(k, acc, rows=rows):
                    ach = attn_v[pl.ds(k * LANES, LANES)]
                    dbase = k * LANES
                    for j in range(LANES):
                        a = ach[j]
                        dcol = jnp.broadcast_to(dbase + j, (LANES,))
                        vs = plsc.load_gather(fsr, [rows, dcol])
                        vd = plsc.load_gather(fdr, [rows, dcol])
                        u = vs + vd
                        acc = acc + u * (0.6 * a) + jnp.abs(u) * (0.4 * a)
                    return acc

                acc = lax.fori_loop(0, D // LANES, dot_body,
                                    jnp.zeros((LANES,), jnp.float32))
                exv = jnp.exp(acc)
                ex_all[pl.ds(base + g * LANES, LANES)] = exv
                plsc.addupdate_scatter(den_priv, [dstg], exv)

        issue(0, fsr0, fdr0, s0)

        def pair_body(k, _):
            c0 = k * 2
            c1 = c0 + 1
            issue(c1, fsr1, fdr1, s1)
            waitg(c0, fsr0, fdr0, s0)
            compute(c0, fsr0, fdr0)
            issue(c0 + 2, fsr0, fdr0, s0)
            waitg(c1, fsr1, fdr1, s1)
            compute(c1, fsr1, fdr1)
            return 0

        lax.fori_loop(0, (NCHUNK - 1) // 2, pair_body, 0)
        waitg(NCHUNK - 1, fsr0, fdr0, s0)
        compute(NCHUNK - 1, fsr0, fdr0)

        pltpu.sync_copy(ex_all, ex_hbm.at[pl.ds(woff, EW)])
        pltpu.sync_copy(den_priv, den_all_hbm.at[wid])

    return pass1


def _make_pass2(N, E, NPAD, C2):
    EW = E // NW
    NCHUNK = EW // C2
    mesh = plsc.VectorSubcoreMesh(core_axis_name="c", subcore_axis_name="s",
                                  num_cores=NC, num_subcores=NS)

    @functools.partial(
        pl.kernel,
        out_type=[jax.ShapeDtypeStruct((NW, NPAD), jnp.float32)],
        mesh=mesh,
        compiler_params=pltpu.CompilerParams(needs_layout_passes=False),
        scratch_types=[
            pltpu.VMEM((EW,), jnp.int32),
            pltpu.VMEM((EW,), jnp.int32),
            pltpu.VMEM((EW,), jnp.float32),
            pltpu.VMEM((NPAD,), jnp.float32),
            pltpu.VMEM((NPAD,), jnp.float32),
            pltpu.VMEM((NPAD,), jnp.float32),
            pltpu.SemaphoreType.DMA,
        ],
    )
    def pass2(ex_hbm, esrc_hbm, edst_hbm, den_all_hbm, w_all_hbm,
              src_all, dst_all, ex_all, den_v, tmp_v, w_priv, sem):
        wid = lax.axis_index("s") * NC + lax.axis_index("c")
        woff = wid * EW
        nvec = NPAD // LANES

        cps = pltpu.async_copy(esrc_hbm.at[pl.ds(woff, EW)], src_all, sem)
        cpd = pltpu.async_copy(edst_hbm.at[pl.ds(woff, EW)], dst_all, sem)
        cpe = pltpu.async_copy(ex_hbm.at[pl.ds(woff, EW)], ex_all, sem)

        def zero_body(i, _):
            w_priv[pl.ds(i * LANES, LANES)] = jnp.zeros((LANES,), jnp.float32)
            den_v[pl.ds(i * LANES, LANES)] = jnp.zeros((LANES,), jnp.float32)
            return 0
        lax.fori_loop(0, nvec, zero_body, 0)

        def acc_row(r, _):
            pltpu.sync_copy(den_all_hbm.at[r], tmp_v)

            def add_body(i, _):
                sl = pl.ds(i * LANES, LANES)
                den_v[sl] = den_v[sl] + tmp_v[sl]
                return 0
            lax.fori_loop(0, nvec, add_body, 0)
            return 0
        lax.fori_loop(0, NW, acc_row, 0)

        cps.wait()
        cpd.wait()
        cpe.wait()

        def grp_body(g, _):
            sl = pl.ds(g * LANES, LANES)
            dstg = dst_all[sl]
            srcg = src_all[sl]
            exg = ex_all[sl]
            dv = plsc.load_gather(den_v, [dstg])
            a = exg / (dv + 1e-16)
            plsc.addupdate_scatter(w_priv, [srcg], a)
            return 0
        lax.fori_loop(0, EW // LANES, grp_body, 0)
        pltpu.sync_copy(w_priv, w_all_hbm.at[wid])

    return pass2


def _make_final(N, D, NPAD):
    def _final_body(w_all_ref, fs_ref, gb_ref, w1_ref, b1_ref, w2_ref, b2_ref,
                    out_ref):
        w = jnp.sum(w_all_ref[...], axis=0)[:N]
        pooled = jnp.sum(fs_ref[...] * w[:, None], axis=0) * (1.0 / N)
        hg = jax.nn.sigmoid(pooled + gb_ref[0])
        h1 = jnp.sum(w1_ref[...] * hg[:, None], axis=0) + b1_ref[0]
        h2 = jnp.sum(w2_ref[...] * h1[:, None], axis=0) + b2_ref[0]
        out_ref[...] = jax.nn.sigmoid(h2).reshape(1, 1)

    def _final(w_all, fs, gb, W1, b1, W2, b2):
        return pl.pallas_call(
            _final_body,
            out_shape=jax.ShapeDtypeStruct((1, 1), jnp.float32),
        )(w_all, fs, gb, W1, b1, W2, b2)

    return _final


def kernel(x, edge_index, Wsrc, bsrc, Wdst, bdst, attn, gat_bias, W1, b1,
           W2, b2):
    N, D = x.shape
    E = edge_index.shape[1]
    NPAD = 10240
    C = 80
    C2 = 2000

    fs, fd = _proj(x, Wsrc, bsrc.reshape(1, -1), Wdst, bdst.reshape(1, -1))
    esrc = edge_index[0]
    edst = edge_index[1]
    ex, den_all = _make_pass1(N, E, D, NPAD, C)(fs, fd, esrc, edst,
                                                attn.reshape(-1))
    w_all, = _make_pass2(N, E, NPAD, C2)(ex, esrc, edst, den_all)
    out = _make_final(N, D, NPAD)(w_all, fs, gat_bias.reshape(1, -1),
                                  W1, b1.reshape(1, -1), W2, b2.reshape(1, -1))
    return out.reshape(1, 1, 1)


# speedup vs baseline: 5.7186x; 1.0859x over previous
import functools

import jax
import jax.numpy as jnp
from jax import lax
from jax.experimental import pallas as pl
from jax.experimental.pallas import tpu as pltpu
from jax.experimental.pallas import tpu_sc as plsc

NC = 2
NS = 16
NW = NC * NS
LANES = 16


def _proj_body(x_ref, ws_ref, bs_ref, wd_ref, bd_ref, fs_ref, fd_ref):
    x = x_ref[...]
    fs_ref[...] = jnp.dot(x, ws_ref[...],
                          preferred_element_type=jnp.float32) + bs_ref[...]
    fd_ref[...] = jnp.dot(x, wd_ref[...],
                          preferred_element_type=jnp.float32) + bd_ref[...]


def _proj(x, Ws, bs, Wd, bd):
    n, d = x.shape
    return pl.pallas_call(
        _proj_body,
        out_shape=[jax.ShapeDtypeStruct((n, d), jnp.float32),
                   jax.ShapeDtypeStruct((n, d), jnp.float32)],
    )(x, Ws, bs, Wd, bd)


def _make_pass1(N, E, D, NPAD, C):
    EW = E // NW
    NCHUNK = EW // C
    G = C // LANES
    mesh = plsc.VectorSubcoreMesh(core_axis_name="c", subcore_axis_name="s",
                                  num_cores=NC, num_subcores=NS)

    NROW = NPAD // D

    @functools.partial(
        pl.kernel,
        out_type=[jax.ShapeDtypeStruct((E,), jnp.float32),
                  jax.ShapeDtypeStruct((NC, NROW, D), jnp.float32)],
        mesh=mesh,
        compiler_params=pltpu.CompilerParams(needs_layout_passes=False),
        scratch_types=[
            pltpu.VMEM((EW,), jnp.int32),
            pltpu.VMEM((EW,), jnp.int32),
            pltpu.VMEM((C, D), jnp.float32),
            pltpu.VMEM((C, D), jnp.float32),
            pltpu.VMEM((C, D), jnp.float32),
            pltpu.VMEM((C, D), jnp.float32),
            pltpu.VMEM((EW,), jnp.float32),
            pltpu.VMEM((NROW, D), jnp.float32),
            pltpu.VMEM((NROW,), jnp.int32),
            pltpu.VMEM((D,), jnp.float32),
            pltpu.VMEM_SHARED((NROW, D), jnp.float32),
            pltpu.SemaphoreType.DMA,
            pltpu.SemaphoreType.DMA,
        ],
    )
    def pass1(fs_hbm, fd_hbm, esrc_hbm, edst_hbm, attn_hbm, ex_hbm,
              den_all_hbm,
              src_all, dst_all, fsr0, fdr0, fsr1, fdr1, ex_all, den_priv,
              idx640, attn_v, den_sh, s0, s1):
        sid = lax.axis_index("s")
        cid = lax.axis_index("c")
        wid = sid * NC + cid
        woff = wid * EW
        pltpu.sync_copy(attn_hbm, attn_v)
        pltpu.sync_copy(esrc_hbm.at[pl.ds(woff, EW)], src_all)
        pltpu.sync_copy(edst_hbm.at[pl.ds(woff, EW)], dst_all)

        iota = lax.iota(jnp.int32, LANES)

        def zero_body(q, _):
            row = lax.shift_right_logical(q, 3)
            off = jnp.bitwise_and(q, 7) * LANES
            den_priv[row, pl.ds(off, LANES)] = jnp.zeros((LANES,),
                                                         jnp.float32)
            return 0
        lax.fori_loop(0, NROW * (D // LANES), zero_body, 0)

        def idx_body(i, _):
            idx640[pl.ds(i * LANES, LANES)] = iota + i * LANES
            return 0
        lax.fori_loop(0, NROW // LANES, idx_body, 0)

        def issue(c, fsr, fdr, sem):
            sl = pl.ds(c * C, C)
            pltpu.async_copy(fs_hbm.at[src_all.at[sl]], fsr, sem)
            pltpu.async_copy(fd_hbm.at[dst_all.at[sl]], fdr, sem)

        def waitg(c, fsr, fdr, sem):
            sl = pl.ds(c * C, C)
            pltpu.make_async_copy(fs_hbm.at[src_all.at[sl]], fsr, sem).wait()
            pltpu.make_async_copy(fd_hbm.at[dst_all.at[sl]], fdr, sem).wait()

        def compute(c, fsr, fdr):
            base = c * C
            for g in range(G):
                rows = iota + g * LANES
                dstg = dst_all[pl.ds(base + g * LANES, LANES)]

                def dot_body(k, carry, rows=rows):
                    col, acc = carry
                    ach = attn_v[pl.ds(k * LANES, LANES)]
                    for j in range(LANES):
                        a = ach[j]
                        vs = plsc.load_gather(fsr, [rows, col])
                        vd = plsc.load_gather(fdr, [rows, col])
                        u = vs + vd
                        lr = jnp.maximum(u, u * 0.2)
                        acc = acc + lr * a
                        col = col + 1
                    return (col, acc)

                _, acc = lax.fori_loop(
                    0, D // LANES, dot_body,
                    (jnp.zeros((LANES,), jnp.int32),
                     jnp.zeros((LANES,), jnp.float32)))
                exv = jnp.exp(acc)
                ex_all[pl.ds(base + g * LANES, LANES)] = exv
                rowi = lax.shift_right_logical(dstg, 7)
                coli = jnp.bitwise_and(dstg, D - 1)
                plsc.addupdate_scatter(den_priv, [rowi, coli], exv)

        issue(0, fsr0, fdr0, s0)

        def pair_body(k, _):
            c0 = k * 2
            c1 = c0 + 1
            issue(c1, fsr1, fdr1, s1)
            waitg(c0, fsr0, fdr0, s0)
            compute(c0, fsr0, fdr0)
            issue(c0 + 2, fsr0, fdr0, s0)
            waitg(c1, fsr1, fdr1, s1)
            compute(c1, fsr1, fdr1)
            return 0

        lax.fori_loop(0, (NCHUNK - 1) // 2, pair_body, 0)
        waitg(NCHUNK - 1, fsr0, fdr0, s0)
        compute(NCHUNK - 1, fsr0, fdr0)

        pltpu.sync_copy(ex_all, ex_hbm.at[pl.ds(woff, EW)])

        @pl.when(sid == 0)
        def _():
            pltpu.sync_copy(den_priv, den_sh)
        plsc.subcore_barrier()

        @pl.when(sid != 0)
        def _():
            pltpu.sync_copy(den_priv, den_sh.at[idx640], add=True)
        plsc.subcore_barrier()

        @pl.when(sid == 0)
        def _():
            pltpu.sync_copy(den_sh, den_all_hbm.at[cid])

    return pass1


def _make_pass2(N, E, NPAD, D):
    EW = E // NW
    NROW = NPAD // D
    mesh = plsc.VectorSubcoreMesh(core_axis_name="c", subcore_axis_name="s",
                                  num_cores=NC, num_subcores=NS)

    @functools.partial(
        pl.kernel,
        out_type=[jax.ShapeDtypeStruct((NW, NPAD), jnp.float32)],
        mesh=mesh,
        compiler_params=pltpu.CompilerParams(needs_layout_passes=False),
        scratch_types=[
            pltpu.VMEM((EW,), jnp.int32),
            pltpu.VMEM((EW,), jnp.int32),
            pltpu.VMEM((EW,), jnp.float32),
            pltpu.VMEM((NROW, D), jnp.float32),
            pltpu.VMEM((NROW, D), jnp.float32),
            pltpu.VMEM((NPAD,), jnp.float32),
            pltpu.SemaphoreType.DMA,
        ],
    )
    def pass2(ex_hbm, esrc_hbm, edst_hbm, den_all_hbm, w_all_hbm,
              src_all, dst_all, ex_all, den_v, tmp_v, w_priv, sem):
        wid = lax.axis_index("s") * NC + lax.axis_index("c")
        woff = wid * EW

        cps = pltpu.async_copy(esrc_hbm.at[pl.ds(woff, EW)], src_all, sem)
        cpd = pltpu.async_copy(edst_hbm.at[pl.ds(woff, EW)], dst_all, sem)
        cpe = pltpu.async_copy(ex_hbm.at[pl.ds(woff, EW)], ex_all, sem)
        pltpu.sync_copy(den_all_hbm.at[0], den_v)
        pltpu.sync_copy(den_all_hbm.at[1], tmp_v)

        def zero_body(i, _):
            w_priv[pl.ds(i * LANES, LANES)] = jnp.zeros((LANES,), jnp.float32)
            return 0
        lax.fori_loop(0, NPAD // LANES, zero_body, 0)

        def add_body(q, _):
            row = lax.shift_right_logical(q, 3)
            off = jnp.bitwise_and(q, 7) * LANES
            sl = pl.ds(off, LANES)
            den_v[row, sl] = den_v[row, sl] + tmp_v[row, sl]
            return 0
        lax.fori_loop(0, NROW * (D // LANES), add_body, 0)

        cps.wait()
        cpd.wait()
        cpe.wait()

        def grp_body(g, _):
            sl = pl.ds(g * LANES, LANES)
            dstg = dst_all[sl]
            srcg = src_all[sl]
            exg = ex_all[sl]
            rowi = lax.shift_right_logical(dstg, 7)
            coli = jnp.bitwise_and(dstg, D - 1)
            dv = plsc.load_gather(den_v, [rowi, coli])
            a = exg / (dv + 1e-16)
            plsc.addupdate_scatter(w_priv, [srcg], a)
            return 0
        lax.fori_loop(0, EW // LANES, grp_body, 0)
        pltpu.sync_copy(w_priv, w_all_hbm.at[wid])

    return pass2


def _make_final(N, D, NPAD):
    def _final_body(w_all_ref, fs_ref, gb_ref, w1_ref, b1_ref, w2_ref, b2_ref,
                    out_ref):
        w = jnp.sum(w_all_ref[...], axis=0)[:N]
        pooled = jnp.sum(fs_ref[...] * w[:, None], axis=0) * (1.0 / N)
        hg = jax.nn.sigmoid(pooled + gb_ref[0])
        h1 = jnp.sum(w1_ref[...] * hg[:, None], axis=0) + b1_ref[0]
        h2 = jnp.sum(w2_ref[...] * h1[:, None], axis=0) + b2_ref[0]
        out_ref[...] = jax.nn.sigmoid(h2).reshape(1, 1)

    def _final(w_all, fs, gb, W1, b1, W2, b2):
        return pl.pallas_call(
            _final_body,
            out_shape=jax.ShapeDtypeStruct((1, 1), jnp.float32),
        )(w_all, fs, gb, W1, b1, W2, b2)

    return _final


def kernel(x, edge_index, Wsrc, bsrc, Wdst, bdst, attn, gat_bias, W1, b1,
           W2, b2):
    N, D = x.shape
    E = edge_index.shape[1]
    NPAD = 10240
    C = 80

    fs, fd = _proj(x, Wsrc, bsrc.reshape(1, -1), Wdst, bdst.reshape(1, -1))
    esrc = edge_index[0]
    edst = edge_index[1]
    ex, den_all = _make_pass1(N, E, D, NPAD, C)(fs, fd, esrc, edst,
                                                attn.reshape(-1))
    w_all, = _make_pass2(N, E, NPAD, D)(ex, esrc, edst, den_all)
    out = _make_final(N, D, NPAD)(w_all, fs, gat_bias.reshape(1, -1),
                                  W1, b1.reshape(1, -1), W2, b2.reshape(1, -1))
    return out.reshape(1, 1, 1)
